# baseline (speedup 1.0000x reference)
import jax
import jax.numpy as jnp
from jax.experimental import pallas as pl
from jax.experimental.pallas import tpu as pltpu

_A = 32
_F = 128
_K = 16
_L = 3
_H = 128
_SCALE_K = 1.0
_LIMIT = 0.1
_EPS = 1e-9
_G = 4


def _pairwise_geometry(fxi, fyi, fzi, fxj, fyj, fzj, rho):
    pos_i = []
    pos_j = []
    for c in range(3):
        pos_i.append(fxi * rho[0][c] + fyi * rho[1][c] + fzi * rho[2][c])
        pos_j.append(fxj * rho[0][c] + fyj * rho[1][c] + fzj * rho[2][c])
    v = [pos_j[c] - pos_i[c] for c in range(3)]
    d = jnp.sqrt(v[0] * v[0] + v[1] * v[1] + v[2] * v[2] + _EPS)
    return v, d


def _mpnn(h, wbig, Wm, Ws, Wn):
    m = jax.nn.relu(jnp.dot(h, Wm, preferred_element_type=jnp.float32))
    agg = jnp.dot(wbig, m, preferred_element_type=jnp.float32)
    return jax.nn.relu(
        jnp.dot(h, Ws, preferred_element_type=jnp.float32)
        + jnp.dot(agg, Wn, preferred_element_type=jnp.float32))


def _kern(frac_ref, z_ref, emb_ref, mWm_ref, mWs_ref, mWn_ref,
          uWm_ref, uWs_ref, uWn_ref, W1_ref, Wet_ref, out_ref):
    G, A, F, K, H = _G, _A, _F, _K, _H
    GA = G * A
    f32 = jnp.float32

    frac = frac_ref[...]
    fxi = frac[:, 0:1].reshape(G, A, 1)
    fyi = frac[:, 1:2].reshape(G, A, 1)
    fzi = frac[:, 2:3].reshape(G, A, 1)
    fxj = jnp.swapaxes(fxi, 1, 2)
    fyj = jnp.swapaxes(fyi, 1, 2)
    fzj = jnp.swapaxes(fzi, 1, 2)

    dx = fxi - fxj
    dy = fyi - fyj
    dz = fzi - fzj
    d2 = dx * dx + dy * dy + dz * dz
    ii = jax.lax.broadcasted_iota(jnp.int32, (G, A, A), 1)
    jj = jax.lax.broadcasted_iota(jnp.int32, (G, A, A), 2)
    d2 = d2 + jnp.where(ii == jj, f32(1e9), f32(0.0))

    d2_j = d2[:, :, :, None]
    d2_jp = d2[:, :, None, :]
    jp4 = jax.lax.broadcasted_iota(jnp.int32, (G, A, A, A), 3)
    j4 = jax.lax.broadcasted_iota(jnp.int32, (G, A, A, A), 2)
    beats = jnp.where(
        (d2_jp < d2_j) | ((d2_jp == d2_j) & (jp4 < j4)), f32(1.0), f32(0.0))
    rank = jnp.sum(beats, axis=3)
    sel = jnp.where(rank < f32(K), f32(1.0), f32(0.0))

    z = z_ref[...]
    lanes = jax.lax.broadcasted_iota(jnp.int32, (GA, 128), 1)
    onehot = jnp.where(z == lanes, f32(1.0), f32(0.0))
    h = jnp.dot(onehot, emb_ref[...], preferred_element_type=f32)

    rowg = jax.lax.broadcasted_iota(jnp.int32, (GA, GA), 0) // A
    colg = jax.lax.broadcasted_iota(jnp.int32, (GA, GA), 1) // A
    bdmask = jnp.where(rowg == colg, f32(1.0), f32(0.0))

    def make_wbig(wadj):
        wr = wadj.reshape(GA, A)
        wtile = jnp.concatenate([wr] * G, axis=1)
        return wtile * bdmask

    rho = [[jnp.full((G, 1, 1), f32(1.0) if a == c else f32(0.0))
            for c in range(3)] for a in range(3)]

    v, d = _pairwise_geometry(fxi, fyi, fzi, fxj, fyj, fzj, rho)
    wbig = make_wbig(sel * jnp.exp(-d))

    for l in range(_L):
        h = _mpnn(h, wbig, mWm_ref[l], mWs_ref[l], mWn_ref[l])

    AA = A * A
    al = jax.lax.broadcasted_iota(jnp.int32, (A, AA), 1)
    asub = jax.lax.broadcasted_iota(jnp.int32, (A, AA), 0)
    RJ = jnp.where(al // A == asub, f32(1.0), f32(0.0))
    RK = jnp.where(al % A == asub, f32(1.0), f32(0.0))

    def rep_j(fd):
        return jnp.dot(fd.reshape(GA, A), RJ,
                       preferred_element_type=f32).reshape(G, A, AA)

    def rep_k(fd):
        return jnp.dot(fd.reshape(GA, A), RK,
                       preferred_element_type=f32).reshape(G, A, AA)

    neq_l = jax.lax.broadcasted_iota(jnp.int32, (1, 1, AA), 2)
    neq = jnp.where(neq_l // A == neq_l % A, f32(0.0), f32(1.0))
    selj = rep_j(sel)
    selk = rep_k(sel)
    selpair = selj * selk * neq

    for l in range(_L):
        h = _mpnn(h, wbig, uWm_ref[l], uWs_ref[l], uWn_ref[l])

        P = jnp.dot(h, W1_ref[l], preferred_element_type=f32)
        P3 = P.reshape(G, A, H)
        S = jax.nn.relu(P3[:, :, None, :] + P3[:, None, :, :])
        weH = Wet_ref[2 * l:2 * l + 1, :].reshape(1, 1, 1, H)
        wtH = Wet_ref[2 * l + 1:2 * l + 2, :].reshape(1, 1, 1, H)
        ew = jnp.tanh(jnp.sum(S * weH, axis=3) * _SCALE_K) * _LIMIT
        tw = jnp.tanh(jnp.sum(S * wtH, axis=3) * _SCALE_K) * _LIMIT

        vh = [v[c] / (d + _EPS) for c in range(3)]

        mew = sel * ew
        inv_ak = f32(1.0 / (A * K))
        et = [[jnp.sum(mew * vh[p] * vh[q], axis=(1, 2), keepdims=True)
               * inv_ak for q in range(3)] for p in range(3)]

        vj = [rep_j(vh[c]) for c in range(3)]
        vk = [rep_k(vh[c]) for c in range(3)]
        cx = vj[1] * vk[2] - vj[2] * vk[1]
        cy = vj[2] * vk[0] - vj[0] * vk[2]
        cz = vj[0] * vk[1] - vj[1] * vk[0]
        sin = jnp.sqrt(cx * cx + cy * cy + cz * cz + _EPS)
        mask = jnp.where(sin > f32(1e-3), f32(1.0), f32(0.0)) * selpair
        inv_sin = f32(1.0) / (sin + _EPS)
        ch = [cx * inv_sin, cy * inv_sin, cz * inv_sin]
        tpw = rep_j(tw) * rep_k(tw) * mask
        denom = f32(1.0) / (jnp.sum(mask, axis=(1, 2), keepdims=True)
                            + f32(1.0))
        tt = [[jnp.sum(tpw * ch[p] * ch[q], axis=(1, 2),
                       keepdims=True) * denom
               for q in range(3)] for p in range(3)]

        act = [[et[p][q] + tt[p][q] + (f32(1.0) if p == q else f32(0.0))
                for q in range(3)] for p in range(3)]
        rho = [[act[p][0] * rho[0][q] + act[p][1] * rho[1][q]
                + act[p][2] * rho[2][q] for q in range(3)] for p in range(3)]

        if l + 1 < _L:
            v, d = _pairwise_geometry(fxi, fyi, fzi, fxj, fyj, fzj, rho)
            wbig = make_wbig(sel * jnp.exp(-d))

    out = jnp.concatenate(
        [rho[p][q].reshape(G, 1) for p in range(3) for q in range(3)], axis=1)
    out_ref[...] = out.reshape(1, G, 9)


def kernel(cell, x, z, struct_size, embedding, mpnn_Wm, mpnn_Ws, mpnn_Wn,
           upd_Wm, upd_Ws, upd_Wn, act_W1, act_We, act_Wt):
    del cell, struct_size
    f32 = jnp.float32
    N = x.shape[0]
    Bv = N // _A
    frac = jnp.mod(x, 1.0).astype(f32)
    z2 = z.astype(jnp.int32).reshape(N, 1)
    embp = jnp.zeros((128, _F), f32).at[:embedding.shape[0]].set(
        embedding.astype(f32))
    wet = jnp.concatenate([act_We, act_Wt], axis=2)
    wet = jnp.transpose(wet, (0, 2, 1)).reshape(2 * _L, _H)
    wet = jnp.zeros((8, _H), f32).at[:2 * _L].set(wet)

    grid = (Bv // _G,)
    GA = _G * _A
    full = lambda shape: pl.BlockSpec(shape, lambda b: tuple(0 for _ in shape))
    out = pl.pallas_call(
        _kern,
        grid=grid,
        in_specs=[
            pl.BlockSpec((GA, 3), lambda b: (b, 0)),
            pl.BlockSpec((GA, 1), lambda b: (b, 0)),
            full((128, _F)),
            full((_L, _F, _F)), full((_L, _F, _F)), full((_L, _F, _F)),
            full((_L, _F, _F)), full((_L, _F, _F)), full((_L, _F, _F)),
            full((_L, _F, _H)),
            full((8, _H)),
        ],
        out_specs=pl.BlockSpec((1, _G, 9), lambda b: (b, 0, 0)),
        out_shape=jax.ShapeDtypeStruct((Bv // _G, _G, 9), f32),
    )(frac, z2, embp,
      mpnn_Wm.astype(f32), mpnn_Ws.astype(f32), mpnn_Wn.astype(f32),
      upd_Wm.astype(f32), upd_Ws.astype(f32), upd_Wn.astype(f32),
      act_W1.astype(f32), wet)
    return out.reshape(Bv, 3, 3)


# speedup vs baseline: 1.2032x; 1.2032x over previous
import jax
import jax.numpy as jnp
from jax.experimental import pallas as pl
from jax.experimental.pallas import tpu as pltpu

_A = 32
_F = 128
_K = 16
_L = 3
_H = 128
_SCALE_K = 1.0
_LIMIT = 0.1
_EPS = 1e-9
_G = 8


def _pairwise_geometry(fxi, fyi, fzi, fxj, fyj, fzj, rho):
    pos_i = []
    pos_j = []
    for c in range(3):
        pos_i.append(fxi * rho[0][c] + fyi * rho[1][c] + fzi * rho[2][c])
        pos_j.append(fxj * rho[0][c] + fyj * rho[1][c] + fzj * rho[2][c])
    v = [pos_j[c] - pos_i[c] for c in range(3)]
    d = jnp.sqrt(v[0] * v[0] + v[1] * v[1] + v[2] * v[2] + _EPS)
    return v, d


def _mpnn(h, wadj, Wm, Ws, Wn):
    G, A, F = wadj.shape[0], wadj.shape[1], h.shape[1]
    m = jax.nn.relu(jnp.dot(h, Wm, preferred_element_type=jnp.float32))
    agg = jax.lax.dot_general(
        wadj, m.reshape(G, A, F), (((2,), (1,)), ((0,), (0,))),
        preferred_element_type=jnp.float32).reshape(G * A, F)
    return jax.nn.relu(
        jnp.dot(h, Ws, preferred_element_type=jnp.float32)
        + jnp.dot(agg, Wn, preferred_element_type=jnp.float32))


def _kern(frac_ref, z_ref, emb_ref, mWm_ref, mWs_ref, mWn_ref,
          uWm_ref, uWs_ref, uWn_ref, W1_ref, Wet_ref, out_ref):
    G, A, F, K, H = _G, _A, _F, _K, _H
    GA = G * A
    f32 = jnp.float32

    frac = frac_ref[...]
    fxi = frac[:, 0:1].reshape(G, A, 1)
    fyi = frac[:, 1:2].reshape(G, A, 1)
    fzi = frac[:, 2:3].reshape(G, A, 1)
    fxj = jnp.swapaxes(fxi, 1, 2)
    fyj = jnp.swapaxes(fyi, 1, 2)
    fzj = jnp.swapaxes(fzi, 1, 2)

    dx = fxi - fxj
    dy = fyi - fyj
    dz = fzi - fzj
    d2 = dx * dx + dy * dy + dz * dz
    ii = jax.lax.broadcasted_iota(jnp.int32, (G, A, A), 1)
    jj = jax.lax.broadcasted_iota(jnp.int32, (G, A, A), 2)
    d2 = d2 + jnp.where(ii == jj, f32(1e9), f32(0.0))

    d2_j = d2[:, :, :, None]
    d2_jp = d2[:, :, None, :]
    jp4 = jax.lax.broadcasted_iota(jnp.int32, (G, A, A, A), 3)
    j4 = jax.lax.broadcasted_iota(jnp.int32, (G, A, A, A), 2)
    beats = jnp.where(
        (d2_jp < d2_j) | ((d2_jp == d2_j) & (jp4 < j4)), f32(1.0), f32(0.0))
    rank = jnp.sum(beats, axis=3)
    sel = jnp.where(rank < f32(K), f32(1.0), f32(0.0))

    z = z_ref[...]
    lanes = jax.lax.broadcasted_iota(jnp.int32, (GA, 128), 1)
    onehot = jnp.where(z == lanes, f32(1.0), f32(0.0))
    h = jnp.dot(onehot, emb_ref[...], preferred_element_type=f32)

    rho = [[jnp.full((G, 1, 1), f32(1.0) if a == c else f32(0.0))
            for c in range(3)] for a in range(3)]

    v, d = _pairwise_geometry(fxi, fyi, fzi, fxj, fyj, fzj, rho)
    wbig = sel * jnp.exp(-d)

    for l in range(_L):
        h = _mpnn(h, wbig, mWm_ref[l], mWs_ref[l], mWn_ref[l])

    AA = A * A
    al = jax.lax.broadcasted_iota(jnp.int32, (A, AA), 1)
    asub = jax.lax.broadcasted_iota(jnp.int32, (A, AA), 0)
    RJ = jnp.where(al // A == asub, f32(1.0), f32(0.0))
    RK = jnp.where(al % A == asub, f32(1.0), f32(0.0))

    def rep_j(fd):
        return jnp.dot(fd.reshape(GA, A), RJ,
                       preferred_element_type=f32).reshape(G, A, AA)

    def rep_k(fd):
        return jnp.dot(fd.reshape(GA, A), RK,
                       preferred_element_type=f32).reshape(G, A, AA)

    neq_l = jax.lax.broadcasted_iota(jnp.int32, (1, 1, AA), 2)
    neq = jnp.where(neq_l // A == neq_l % A, f32(0.0), f32(1.0))
    selj = rep_j(sel)
    selk = rep_k(sel)
    selpair = selj * selk * neq

    for l in range(_L):
        h = _mpnn(h, wbig, uWm_ref[l], uWs_ref[l], uWn_ref[l])

        P = jnp.dot(h, W1_ref[l], preferred_element_type=f32)
        P3 = P.reshape(G, A, H)
        S = jax.nn.relu(P3[:, :, None, :] + P3[:, None, :, :])
        weH = Wet_ref[2 * l:2 * l + 1, :].reshape(1, 1, 1, H)
        wtH = Wet_ref[2 * l + 1:2 * l + 2, :].reshape(1, 1, 1, H)
        ew = jnp.tanh(jnp.sum(S * weH, axis=3) * _SCALE_K) * _LIMIT
        tw = jnp.tanh(jnp.sum(S * wtH, axis=3) * _SCALE_K) * _LIMIT

        vh = [v[c] / (d + _EPS) for c in range(3)]

        mew = sel * ew
        inv_ak = f32(1.0 / (A * K))
        et = [[jnp.sum(mew * vh[p] * vh[q], axis=(1, 2), keepdims=True)
               * inv_ak for q in range(3)] for p in range(3)]

        vj = [rep_j(vh[c]) for c in range(3)]
        vk = [rep_k(vh[c]) for c in range(3)]
        cx = vj[1] * vk[2] - vj[2] * vk[1]
        cy = vj[2] * vk[0] - vj[0] * vk[2]
        cz = vj[0] * vk[1] - vj[1] * vk[0]
        sin = jnp.sqrt(cx * cx + cy * cy + cz * cz + _EPS)
        mask = jnp.where(sin > f32(1e-3), f32(1.0), f32(0.0)) * selpair
        inv_sin = f32(1.0) / (sin + _EPS)
        ch = [cx * inv_sin, cy * inv_sin, cz * inv_sin]
        tpw = rep_j(tw) * rep_k(tw) * mask
        denom = f32(1.0) / (jnp.sum(mask, axis=(1, 2), keepdims=True)
                            + f32(1.0))
        tt = [[jnp.sum(tpw * ch[p] * ch[q], axis=(1, 2),
                       keepdims=True) * denom
               for q in range(3)] for p in range(3)]

        act = [[et[p][q] + tt[p][q] + (f32(1.0) if p == q else f32(0.0))
                for q in range(3)] for p in range(3)]
        rho = [[act[p][0] * rho[0][q] + act[p][1] * rho[1][q]
                + act[p][2] * rho[2][q] for q in range(3)] for p in range(3)]

        if l + 1 < _L:
            v, d = _pairwise_geometry(fxi, fyi, fzi, fxj, fyj, fzj, rho)
            wbig = sel * jnp.exp(-d)

    out = jnp.concatenate(
        [rho[p][q].reshape(G, 1) for p in range(3) for q in range(3)], axis=1)
    out_ref[...] = out.reshape(1, G, 9)


def kernel(cell, x, z, struct_size, embedding, mpnn_Wm, mpnn_Ws, mpnn_Wn,
           upd_Wm, upd_Ws, upd_Wn, act_W1, act_We, act_Wt):
    del cell, struct_size
    f32 = jnp.float32
    N = x.shape[0]
    Bv = N // _A
    frac = jnp.mod(x, 1.0).astype(f32)
    z2 = z.astype(jnp.int32).reshape(N, 1)
    embp = jnp.zeros((128, _F), f32).at[:embedding.shape[0]].set(
        embedding.astype(f32))
    wet = jnp.concatenate([act_We, act_Wt], axis=2)
    wet = jnp.transpose(wet, (0, 2, 1)).reshape(2 * _L, _H)
    wet = jnp.zeros((8, _H), f32).at[:2 * _L].set(wet)

    grid = (Bv // _G,)
    GA = _G * _A
    full = lambda shape: pl.BlockSpec(shape, lambda b: tuple(0 for _ in shape))
    out = pl.pallas_call(
        _kern,
        grid=grid,
        in_specs=[
            pl.BlockSpec((GA, 3), lambda b: (b, 0)),
            pl.BlockSpec((GA, 1), lambda b: (b, 0)),
            full((128, _F)),
            full((_L, _F, _F)), full((_L, _F, _F)), full((_L, _F, _F)),
            full((_L, _F, _F)), full((_L, _F, _F)), full((_L, _F, _F)),
            full((_L, _F, _H)),
            full((8, _H)),
        ],
        out_specs=pl.BlockSpec((1, _G, 9), lambda b: (b, 0, 0)),
        out_shape=jax.ShapeDtypeStruct((Bv // _G, _G, 9), f32),
    )(frac, z2, embp,
      mpnn_Wm.astype(f32), mpnn_Ws.astype(f32), mpnn_Wn.astype(f32),
      upd_Wm.astype(f32), upd_Ws.astype(f32), upd_Wn.astype(f32),
      act_W1.astype(f32), wet)
    return out.reshape(Bv, 3, 3)
